# Initial kernel scaffold; baseline (speedup 1.0000x reference)
import jax
import jax.numpy as jnp
from jax.experimental import pallas as pl


def kernel(tokens, fingerprints, Wq, Wk, Wv, alpha, gW1, gb1, gW2, gb2, penta, fusion_w, temperature):
    raise NotImplementedError("write your pallas kernel here")


# speedup vs baseline: 4.5718x; 4.5718x over previous
import functools

import jax
import jax.numpy as jnp
from jax import lax
from jax.experimental import pallas as pl
from jax.experimental.pallas import tpu as pltpu

E = 16
D = 2048
DE = 128
B = 4
P = 2048
S = D // E
H = S // 4
PBLK = 256
LOG2S = 7


def _stage1_body(tok_ref, fp_ref, gW1c_ref, gb1_ref, gW2m_ref, gb2_ref,
                 alpha_ref, wq_ref, wk_ref, wv_ref, penta_ref,
                 qa_ref, ka_ref, v_ref):
    tok = tok_ref[0]
    fp = fp_ref[...]
    a = jnp.clip(jnp.floor(fp * E).astype(jnp.int32), 0, E - 1)
    eidx = lax.broadcasted_iota(jnp.int32, (PBLK, E), 1)
    onehot = (eidx == a).astype(jnp.float32)
    dcol = lax.broadcasted_iota(jnp.int32, (PBLK, D), 1)
    mask = (lax.shift_right_logical(dcol, LOG2S) == a).astype(jnp.float32)
    m = tok * mask
    h = jax.nn.gelu(jnp.dot(m, gW1c_ref[...]) + jnp.dot(onehot, gb1_ref[...]))
    gpre = (jnp.sum(h * jnp.dot(onehot, gW2m_ref[...]), axis=-1, keepdims=True)
            + jnp.dot(onehot, gb2_ref[...]))
    g = jax.nn.sigmoid(gpre)
    aw = jnp.dot(onehot, jax.nn.sigmoid(alpha_ref[...]))
    u = m * (g * aw + (1.0 - aw))
    q = jnp.dot(u, wq_ref[...])
    k = jnp.dot(u, wk_ref[...])
    v = jnp.dot(u, wv_ref[...])
    v_ref[0] = v
    kas, qas = [], []
    for vtx in range(5):
        pv = penta_ref[vtx]
        nrm = jnp.sqrt(jnp.sum(pv * pv, axis=-1, keepdims=True))
        dv = pv / (nrm + 1e-8)
        dsel = jnp.dot(onehot, dv)
        kas.append(jnp.sum(k * dsel, axis=-1, keepdims=True))
        qas.append(jnp.sum(q * dsel, axis=-1, keepdims=True))
    ka_ref[0] = jnp.concatenate(kas, axis=1)
    qa_ref[0] = jnp.concatenate(qas, axis=1)


def _stage2_body(ka_ref, qa_ref, v_ref, fw_ref, temp_ref, out_ref):
    ka = ka_ref[0] / temp_ref[0, 0]
    mx = jnp.max(ka, axis=0, keepdims=True)
    ex = jnp.exp(ka - mx)
    w = ex / jnp.sum(ex, axis=0, keepdims=True)
    ctx = lax.dot_general(w, v_ref[0], (((0,), (0,)), ((), ())))
    qf = qa_ref[0] * fw_ref[...]
    out_ref[0] = jnp.dot(qf, ctx)


@jax.jit
def kernel(tokens, fingerprints, Wq, Wk, Wv, alpha, gW1, gb1, gW2, gb2,
           penta, fusion_w, temperature):
    gW1c = gW1.reshape(E * S, H)
    wqc = Wq.reshape(E * S, DE)
    wkc = Wk.reshape(E * S, DE)
    wvc = Wv.reshape(E * S, DE)
    gW2m = gW2[:, :, 0]
    alpha2 = alpha.reshape(E, 1)
    penta_vm = penta.transpose(1, 0, 2)
    fp2 = fingerprints.reshape(P, 1)
    fw2 = fusion_w.reshape(1, 5)
    temp2 = temperature.reshape(1, 1)

    nblk = P // PBLK
    full = lambda i, j: (0, 0)
    qa, ka, v = pl.pallas_call(
        _stage1_body,
        grid=(B, nblk),
        in_specs=[
            pl.BlockSpec((1, PBLK, D), lambda b, pb: (b, pb, 0)),
            pl.BlockSpec((PBLK, 1), lambda b, pb: (pb, 0)),
            pl.BlockSpec((E * S, H), full),
            pl.BlockSpec((E, H), full),
            pl.BlockSpec((E, H), full),
            pl.BlockSpec((E, 1), full),
            pl.BlockSpec((E, 1), full),
            pl.BlockSpec((E * S, DE), full),
            pl.BlockSpec((E * S, DE), full),
            pl.BlockSpec((E * S, DE), full),
            pl.BlockSpec((5, E, DE), lambda b, pb: (0, 0, 0)),
        ],
        out_specs=[
            pl.BlockSpec((1, PBLK, 5), lambda b, pb: (b, pb, 0)),
            pl.BlockSpec((1, PBLK, 5), lambda b, pb: (b, pb, 0)),
            pl.BlockSpec((1, PBLK, DE), lambda b, pb: (b, pb, 0)),
        ],
        out_shape=[
            jax.ShapeDtypeStruct((B, P, 5), jnp.float32),
            jax.ShapeDtypeStruct((B, P, 5), jnp.float32),
            jax.ShapeDtypeStruct((B, P, DE), jnp.float32),
        ],
        compiler_params=pltpu.CompilerParams(
            dimension_semantics=("parallel", "parallel")),
    )(tokens, fp2, gW1c, gb1, gW2m, gb2, alpha2, wqc, wkc, wvc, penta_vm)

    out = pl.pallas_call(
        _stage2_body,
        grid=(B,),
        in_specs=[
            pl.BlockSpec((1, P, 5), lambda b: (b, 0, 0)),
            pl.BlockSpec((1, P, 5), lambda b: (b, 0, 0)),
            pl.BlockSpec((1, P, DE), lambda b: (b, 0, 0)),
            pl.BlockSpec((1, 5), lambda b: (0, 0)),
            pl.BlockSpec((1, 1), lambda b: (0, 0)),
        ],
        out_specs=pl.BlockSpec((1, P, DE), lambda b: (b, 0, 0)),
        out_shape=jax.ShapeDtypeStruct((B, P, DE), jnp.float32),
        compiler_params=pltpu.CompilerParams(
            dimension_semantics=("parallel",)),
    )(ka, qa, v, fw2, temp2)
    return out
